# Initial kernel scaffold; baseline (speedup 1.0000x reference)
import jax
import jax.numpy as jnp
from jax.experimental import pallas as pl


def kernel(x, edge_index, W1, b1, W2, b2):
    raise NotImplementedError("write your pallas kernel here")


# speedup vs baseline: 16.3659x; 16.3659x over previous
import functools

import jax
import jax.numpy as jnp
from jax import lax
from jax.experimental import pallas as pl
from jax.experimental.pallas import tpu as pltpu
from jax.experimental.pallas import tpu_sc as plsc

N_NODES = 10000
N_EDGES = 160000
N_PAD = 10240
CHUNK = 125
ROWS = N_EDGES // CHUNK
NC = 2
NS = 16

_MESH = plsc.VectorSubcoreMesh(core_axis_name="c", subcore_axis_name="s")
_SC_PARAMS = pltpu.CompilerParams(use_tc_tiling_on_sc=False)


@functools.partial(
    pl.kernel,
    mesh=_MESH,
    compiler_params=_SC_PARAMS,
    out_type=jax.ShapeDtypeStruct((N_PAD,), jnp.float32),
    scratch_types=[
        pltpu.VMEM_SHARED((N_PAD,), jnp.float32),
        pltpu.VMEM((80, CHUNK), jnp.int32),
        pltpu.VMEM((CHUNK,), jnp.float32),
        pltpu.VMEM((640,), jnp.float32),
    ],
)
def _k1_deg(dst_hbm, ones_hbm, z640_hbm, deg_out, deg_sh, dstv, onesv, zv):
    c = lax.axis_index("c")
    s = lax.axis_index("s")
    pltpu.sync_copy(z640_hbm, zv)
    pltpu.sync_copy(zv, deg_sh.at[pl.ds(640 * s, 640)])
    pltpu.sync_copy(ones_hbm, onesv)
    pltpu.sync_copy(dst_hbm.at[pl.ds(80 * s, 80)], dstv)
    plsc.subcore_barrier()

    def step(j, carry):
        pltpu.sync_copy(onesv, deg_sh.at[dstv.at[j]], add=True)
        return carry

    lax.fori_loop(0, 80, step, 0)
    plsc.subcore_barrier()

    base = 5120 * c + 320 * s
    pltpu.sync_copy(deg_sh.at[pl.ds(base, 320)], zv.at[pl.ds(0, 320)])
    pltpu.sync_copy(zv.at[pl.ds(0, 320)], deg_out.at[pl.ds(base, 320)])


def _make_agg(width):
    @functools.partial(
        pl.kernel,
        mesh=_MESH,
        compiler_params=_SC_PARAMS,
        out_type=jax.ShapeDtypeStruct((NC, N_NODES, width), jnp.float32),
        scratch_types=[
            pltpu.VMEM_SHARED((N_NODES, width), jnp.float32),
            pltpu.VMEM((40, CHUNK), jnp.int32),
            pltpu.VMEM((40, CHUNK), jnp.int32),
            pltpu.VMEM((CHUNK, width), jnp.float32),
            pltpu.SemaphoreType.DMA,
        ],
    )
    def agg(u_hbm, src_hbm, dst_hbm, zrows_hbm, p_out, acc_sh, srcv, dstv,
            rows, sem):
        c = lax.axis_index("c")
        s = lax.axis_index("s")
        pltpu.sync_copy(zrows_hbm, rows)
        for k in range(5):
            pltpu.sync_copy(rows, acc_sh.at[pl.ds(625 * s + 125 * k, 125)])
        rowbase = 640 * c + 40 * s
        pltpu.sync_copy(src_hbm.at[pl.ds(rowbase, 40)], srcv)
        pltpu.sync_copy(dst_hbm.at[pl.ds(rowbase, 40)], dstv)
        plsc.subcore_barrier()

        def step(j, carry):
            pltpu.async_copy(u_hbm.at[srcv.at[j]], rows, sem).wait()
            pltpu.sync_copy(rows, acc_sh.at[dstv.at[j]], add=True)
            return carry

        lax.fori_loop(0, 40, step, 0)
        plsc.subcore_barrier()
        for k in range(5):
            pltpu.sync_copy(acc_sh.at[pl.ds(625 * s + 125 * k, 125)], rows)
            pltpu.sync_copy(rows, p_out.at[c, pl.ds(625 * s + 125 * k, 125)])

    return agg


_agg128 = _make_agg(128)
_agg16 = _make_agg(16)


def _mm_scale_body(x_ref, w_ref, deg_ref, o_ref):
    dinv = lax.rsqrt(deg_ref[...] + 1.0)
    o_ref[...] = jnp.dot(x_ref[...], w_ref[...],
                         preferred_element_type=jnp.float32) * dinv


def _layer2_body(p0_ref, p1_ref, u1_ref, deg_ref, b1_ref, w2_ref, o_ref):
    dinv = lax.rsqrt(deg_ref[...] + 1.0)
    agg = (p0_ref[...] + p1_ref[...] + u1_ref[...]) * dinv + b1_ref[...]
    h = jnp.maximum(agg, 0.0)
    o_ref[...] = jnp.dot(h, w2_ref[...],
                         preferred_element_type=jnp.float32) * dinv


def _final_body(r0_ref, r1_ref, u2_ref, deg_ref, b2_ref, o_ref):
    dinv = lax.rsqrt(deg_ref[...] + 1.0)
    o_ref[...] = ((r0_ref[...] + r1_ref[...] + u2_ref[...]) * dinv
                  + b2_ref[...])


def kernel(x, edge_index, W1, b1, W2, b2):
    f32 = jnp.float32
    ei = edge_index.astype(jnp.int32)
    src2 = ei[0].reshape(ROWS, CHUNK)
    dst2 = ei[1].reshape(ROWS, CHUNK)

    ones125 = jnp.ones((CHUNK,), f32)
    z640 = jnp.zeros((640,), f32)
    z128 = jnp.zeros((CHUNK, 128), f32)
    z16 = jnp.zeros((CHUNK, 16), f32)

    deg = _k1_deg(dst2, ones125, z640)
    deg2d = deg[:N_NODES, None]

    MB = 400
    grid = (N_NODES // MB,)
    u1 = pl.pallas_call(
        _mm_scale_body,
        grid=grid,
        in_specs=[
            pl.BlockSpec((MB, 500), lambda i: (i, 0)),
            pl.BlockSpec((500, 128), lambda i: (0, 0)),
            pl.BlockSpec((MB, 1), lambda i: (i, 0)),
        ],
        out_specs=pl.BlockSpec((MB, 128), lambda i: (i, 0)),
        out_shape=jax.ShapeDtypeStruct((N_NODES, 128), f32),
    )(x, W1, deg2d)

    p = _agg128(u1, src2, dst2, z128)

    W2pad = jnp.zeros((128, 16), f32).at[:, :3].set(W2)
    u2 = pl.pallas_call(
        _layer2_body,
        grid=grid,
        in_specs=[
            pl.BlockSpec((MB, 128), lambda i: (i, 0)),
            pl.BlockSpec((MB, 128), lambda i: (i, 0)),
            pl.BlockSpec((MB, 128), lambda i: (i, 0)),
            pl.BlockSpec((MB, 1), lambda i: (i, 0)),
            pl.BlockSpec((1, 128), lambda i: (0, 0)),
            pl.BlockSpec((128, 16), lambda i: (0, 0)),
        ],
        out_specs=pl.BlockSpec((MB, 16), lambda i: (i, 0)),
        out_shape=jax.ShapeDtypeStruct((N_NODES, 16), f32),
    )(p[0], p[1], u1, deg2d, b1[None, :], W2pad)

    r = _agg16(u2, src2, dst2, z16)

    b2pad = jnp.zeros((16,), f32).at[:3].set(b2)
    outp = pl.pallas_call(
        _final_body,
        grid=grid,
        in_specs=[
            pl.BlockSpec((MB, 16), lambda i: (i, 0)),
            pl.BlockSpec((MB, 16), lambda i: (i, 0)),
            pl.BlockSpec((MB, 16), lambda i: (i, 0)),
            pl.BlockSpec((MB, 1), lambda i: (i, 0)),
            pl.BlockSpec((1, 16), lambda i: (0, 0)),
        ],
        out_specs=pl.BlockSpec((MB, 16), lambda i: (i, 0)),
        out_shape=jax.ShapeDtypeStruct((N_NODES, 16), f32),
    )(r[0], r[1], u2, deg2d, b2pad[None, :])

    return outp[:, :3]


# speedup vs baseline: 20.8787x; 1.2757x over previous
import functools

import jax
import jax.numpy as jnp
from jax import lax
from jax.experimental import pallas as pl
from jax.experimental.pallas import tpu as pltpu
from jax.experimental.pallas import tpu_sc as plsc

N_NODES = 10000
N_EDGES = 160000
N_PAD = 10240
CHUNK = 125
ROWS = N_EDGES // CHUNK
NC = 2
NS = 16

_MESH = plsc.VectorSubcoreMesh(core_axis_name="c", subcore_axis_name="s")
_SC_PARAMS = pltpu.CompilerParams(use_tc_tiling_on_sc=False)


@functools.partial(
    pl.kernel,
    mesh=_MESH,
    compiler_params=_SC_PARAMS,
    out_type=jax.ShapeDtypeStruct((N_PAD,), jnp.float32),
    scratch_types=[
        pltpu.VMEM_SHARED((N_PAD,), jnp.float32),
        pltpu.VMEM((80, CHUNK), jnp.int32),
        pltpu.VMEM((CHUNK,), jnp.float32),
        pltpu.VMEM((640,), jnp.float32),
    ],
)
def _k1_deg(dst_hbm, ones_hbm, z640_hbm, deg_out, deg_sh, dstv, onesv, zv):
    c = lax.axis_index("c")
    s = lax.axis_index("s")
    pltpu.sync_copy(z640_hbm, zv)
    pltpu.sync_copy(zv, deg_sh.at[pl.ds(640 * s, 640)])
    pltpu.sync_copy(ones_hbm, onesv)
    pltpu.sync_copy(dst_hbm.at[pl.ds(80 * s, 80)], dstv)
    plsc.subcore_barrier()

    def step(j, carry):
        pltpu.sync_copy(onesv, deg_sh.at[dstv.at[j]], add=True)
        return carry

    lax.fori_loop(0, 80, step, 0)
    plsc.subcore_barrier()

    base = 5120 * c + 320 * s
    pltpu.sync_copy(deg_sh.at[pl.ds(base, 320)], zv.at[pl.ds(0, 320)])
    pltpu.sync_copy(zv.at[pl.ds(0, 320)], deg_out.at[pl.ds(base, 320)])


def _make_agg(width):
    @functools.partial(
        pl.kernel,
        mesh=_MESH,
        compiler_params=_SC_PARAMS,
        out_type=jax.ShapeDtypeStruct((NC, N_NODES, width), jnp.float32),
        scratch_types=[
            pltpu.VMEM_SHARED((N_NODES, width), jnp.float32),
            pltpu.VMEM((40, CHUNK), jnp.int32),
            pltpu.VMEM((40, CHUNK), jnp.int32),
            pltpu.VMEM((CHUNK, width), jnp.float32),
            pltpu.VMEM((CHUNK, width), jnp.float32),
            pltpu.SemaphoreType.DMA,
            pltpu.SemaphoreType.DMA,
        ],
    )
    def agg(u_hbm, src_hbm, dst_hbm, zrows_hbm, p_out, acc_sh, srcv, dstv,
            rows0, rows1, sem, sem2):
        c = lax.axis_index("c")
        s = lax.axis_index("s")
        pltpu.sync_copy(zrows_hbm, rows0)
        for k in range(5):
            pltpu.sync_copy(rows0, acc_sh.at[pl.ds(625 * s + 125 * k, 125)])
        rowbase = 640 * c + 40 * s
        pltpu.sync_copy(src_hbm.at[pl.ds(rowbase, 40)], srcv)
        pltpu.sync_copy(dst_hbm.at[pl.ds(rowbase, 40)], dstv)
        plsc.subcore_barrier()

        pltpu.async_copy(u_hbm.at[srcv.at[0]], rows0, sem)

        def step(t, carry):
            j0 = 2 * t
            pltpu.make_async_copy(u_hbm.at[srcv.at[j0]], rows0, sem).wait()
            pltpu.async_copy(u_hbm.at[srcv.at[j0 + 1]], rows1, sem)
            pltpu.sync_copy(rows0, acc_sh.at[dstv.at[j0]], add=True)
            pltpu.make_async_copy(u_hbm.at[srcv.at[j0 + 1]], rows1, sem).wait()

            @pl.when(t < 19)
            def _():
                pltpu.async_copy(u_hbm.at[srcv.at[j0 + 2]], rows0, sem)

            pltpu.sync_copy(rows1, acc_sh.at[dstv.at[j0 + 1]], add=True)
            return carry

        lax.fori_loop(0, 20, step, 0)
        plsc.subcore_barrier()
        pltpu.sync_copy(acc_sh.at[pl.ds(625 * s, 125)], rows0)
        for k in range(5):
            buf = rows0 if k % 2 == 0 else rows1
            nxt = rows1 if k % 2 == 0 else rows0
            out_cp = pltpu.async_copy(
                buf, p_out.at[c, pl.ds(625 * s + 125 * k, 125)], sem)
            if k < 4:
                in_cp = pltpu.async_copy(
                    acc_sh.at[pl.ds(625 * s + 125 * (k + 1), 125)], nxt, sem2)
                in_cp.wait()
            out_cp.wait()

    return agg


_agg128 = _make_agg(128)
_agg16 = _make_agg(16)


def _mm_body(x_ref, w_ref, o_ref):
    o_ref[...] = jnp.dot(x_ref[...], w_ref[...],
                         preferred_element_type=jnp.float32)


def _scale_body(xw_ref, deg_ref, o_ref):
    o_ref[...] = xw_ref[...] * lax.rsqrt(deg_ref[...] + 1.0)


def _layer2_body(p_ref, u1_ref, deg_ref, b1_ref, w2_ref, o_ref):
    dinv = lax.rsqrt(deg_ref[...] + 1.0)
    pp = p_ref[...]
    agg = (pp[0] + pp[1] + u1_ref[...]) * dinv + b1_ref[...]
    h = jnp.maximum(agg, 0.0)
    o_ref[...] = jnp.dot(h, w2_ref[...],
                         preferred_element_type=jnp.float32) * dinv


def _final_body(r_ref, u2_ref, deg_ref, b2_ref, o_ref):
    dinv = lax.rsqrt(deg_ref[...] + 1.0)
    rr = r_ref[...]
    o_ref[...] = (rr[0] + rr[1] + u2_ref[...]) * dinv + b2_ref[...]


def kernel(x, edge_index, W1, b1, W2, b2):
    f32 = jnp.float32
    ei = edge_index.astype(jnp.int32)
    src2 = ei[0].reshape(ROWS, CHUNK)
    dst2 = ei[1].reshape(ROWS, CHUNK)

    ones125 = jnp.ones((CHUNK,), f32)
    z640 = jnp.zeros((640,), f32)
    z128 = jnp.zeros((CHUNK, 128), f32)
    z16 = jnp.zeros((CHUNK, 16), f32)

    xw = pl.pallas_call(
        _mm_body,
        grid=(10,),
        in_specs=[
            pl.BlockSpec((1000, 500), lambda i: (i, 0)),
            pl.BlockSpec((500, 128), lambda i: (0, 0)),
        ],
        out_specs=pl.BlockSpec((1000, 128), lambda i: (i, 0)),
        out_shape=jax.ShapeDtypeStruct((N_NODES, 128), f32),
    )(x, W1)

    deg = _k1_deg(dst2, ones125, z640)
    deg2d = deg[:N_NODES, None]

    u1 = pl.pallas_call(
        _scale_body,
        grid=(5,),
        in_specs=[
            pl.BlockSpec((2000, 128), lambda i: (i, 0)),
            pl.BlockSpec((2000, 1), lambda i: (i, 0)),
        ],
        out_specs=pl.BlockSpec((2000, 128), lambda i: (i, 0)),
        out_shape=jax.ShapeDtypeStruct((N_NODES, 128), f32),
    )(xw, deg2d)

    p = _agg128(u1, src2, dst2, z128)

    W2pad = jnp.zeros((128, 16), f32).at[:, :3].set(W2)
    u2 = pl.pallas_call(
        _layer2_body,
        grid=(10,),
        in_specs=[
            pl.BlockSpec((2, 1000, 128), lambda i: (0, i, 0)),
            pl.BlockSpec((1000, 128), lambda i: (i, 0)),
            pl.BlockSpec((1000, 1), lambda i: (i, 0)),
            pl.BlockSpec((1, 128), lambda i: (0, 0)),
            pl.BlockSpec((128, 16), lambda i: (0, 0)),
        ],
        out_specs=pl.BlockSpec((1000, 16), lambda i: (i, 0)),
        out_shape=jax.ShapeDtypeStruct((N_NODES, 16), f32),
    )(p, u1, deg2d, b1[None, :], W2pad)

    r = _agg16(u2, src2, dst2, z16)

    b2pad = jnp.zeros((16,), f32).at[:3].set(b2)
    outp = pl.pallas_call(
        _final_body,
        grid=(5,),
        in_specs=[
            pl.BlockSpec((2, 2000, 16), lambda i: (0, i, 0)),
            pl.BlockSpec((2000, 16), lambda i: (i, 0)),
            pl.BlockSpec((2000, 1), lambda i: (i, 0)),
            pl.BlockSpec((1, 16), lambda i: (0, 0)),
        ],
        out_specs=pl.BlockSpec((2000, 16), lambda i: (i, 0)),
        out_shape=jax.ShapeDtypeStruct((N_NODES, 16), f32),
    )(r, u2, deg2d, b2pad[None, :])

    return outp[:, :3]


# speedup vs baseline: 21.4403x; 1.0269x over previous
import functools

import jax
import jax.numpy as jnp
from jax import lax
from jax.experimental import pallas as pl
from jax.experimental.pallas import tpu as pltpu
from jax.experimental.pallas import tpu_sc as plsc

N_NODES = 10000
N_EDGES = 160000
N_PAD = 10240
CHUNK = 125
ROWS = N_EDGES // CHUNK
NC = 2
NS = 16
NB = 4

_MESH = plsc.VectorSubcoreMesh(core_axis_name="c", subcore_axis_name="s")
_SC_PARAMS = pltpu.CompilerParams(use_tc_tiling_on_sc=False)


@functools.partial(
    pl.kernel,
    mesh=_MESH,
    compiler_params=_SC_PARAMS,
    out_type=jax.ShapeDtypeStruct((N_PAD,), jnp.float32),
    scratch_types=[
        pltpu.VMEM_SHARED((N_PAD,), jnp.float32),
        pltpu.VMEM((80, CHUNK), jnp.int32),
        pltpu.VMEM((CHUNK,), jnp.float32),
        pltpu.VMEM((640,), jnp.float32),
    ],
)
def _k1_deg(dst_hbm, ones_hbm, z640_hbm, deg_out, deg_sh, dstv, onesv, zv):
    c = lax.axis_index("c")
    s = lax.axis_index("s")
    pltpu.sync_copy(z640_hbm, zv)
    pltpu.sync_copy(zv, deg_sh.at[pl.ds(640 * s, 640)])
    pltpu.sync_copy(ones_hbm, onesv)
    pltpu.sync_copy(dst_hbm.at[pl.ds(80 * s, 80)], dstv)
    plsc.subcore_barrier()

    def step(j, carry):
        pltpu.sync_copy(onesv, deg_sh.at[dstv.at[j]], add=True)
        return carry

    lax.fori_loop(0, 80, step, 0)
    plsc.subcore_barrier()

    base = 5120 * c + 320 * s
    pltpu.sync_copy(deg_sh.at[pl.ds(base, 320)], zv.at[pl.ds(0, 320)])
    pltpu.sync_copy(zv.at[pl.ds(0, 320)], deg_out.at[pl.ds(base, 320)])


def _make_agg(width, feat_split):
    R = 80 if feat_split else 40
    G = R // NB

    scratch = [
        pltpu.VMEM_SHARED((N_NODES, width), jnp.float32),
        pltpu.VMEM((R, CHUNK), jnp.int32),
        pltpu.VMEM((R, CHUNK), jnp.int32),
    ]
    scratch += [pltpu.VMEM((CHUNK, width), jnp.float32) for _ in range(NB)]
    scratch += [pltpu.SemaphoreType.DMA, pltpu.SemaphoreType.DMA]

    @functools.partial(
        pl.kernel,
        mesh=_MESH,
        compiler_params=_SC_PARAMS,
        out_type=jax.ShapeDtypeStruct((NC, N_NODES, width), jnp.float32),
        scratch_types=scratch,
    )
    def agg(u_hbm, src_hbm, dst_hbm, zrows_hbm, p_out, acc_sh, srcv, dstv,
            r0, r1, r2, r3, gsem, ssem):
        rows = [r0, r1, r2, r3]
        c = lax.axis_index("c")
        s = lax.axis_index("s")
        u_ref = u_hbm.at[c] if feat_split else u_hbm
        pltpu.sync_copy(zrows_hbm, r0)
        for k in range(5):
            pltpu.sync_copy(r0, acc_sh.at[pl.ds(625 * s + 125 * k, 125)])
        rowbase = R * s if feat_split else 640 * c + R * s
        pltpu.sync_copy(src_hbm.at[pl.ds(rowbase, R)], srcv)
        pltpu.sync_copy(dst_hbm.at[pl.ds(rowbase, R)], dstv)
        plsc.subcore_barrier()

        def g_start(j, b):
            pltpu.async_copy(u_ref.at[srcv.at[j]], rows[b], gsem)

        def g_wait(j, b):
            pltpu.make_async_copy(u_ref.at[srcv.at[j]], rows[b], gsem).wait()

        def s_start(j, b):
            pltpu.async_copy(rows[b], acc_sh.at[dstv.at[j]], ssem, add=True)

        def s_wait(j, b):
            pltpu.make_async_copy(rows[b], acc_sh.at[dstv.at[j]], ssem).wait()

        g_start(0, 0)
        g_start(1, 1)

        def step(g, carry):
            for b in range(NB):
                j = NB * g + b
                g_wait(j, b)
                s_start(j, b)
                nb = (b + 2) % NB
                if b < 2:
                    @pl.when(g >= 1)
                    def _():
                        s_wait(j - 2, nb)

                    g_start(j + 2, nb)
                else:
                    @pl.when(g <= G - 2)
                    def _():
                        s_wait(j - 2, nb)
                        g_start(j + 2, nb)
            return carry

        lax.fori_loop(0, G, step, 0)
        for b in range(NB):
            s_wait(R - NB + b, b)
        plsc.subcore_barrier()
        pltpu.sync_copy(acc_sh.at[pl.ds(625 * s, 125)], r0)
        for k in range(5):
            buf = rows[k % 2]
            nxt = rows[(k + 1) % 2]
            out_cp = pltpu.async_copy(
                buf, p_out.at[c, pl.ds(625 * s + 125 * k, 125)], gsem)
            if k < 4:
                pltpu.async_copy(
                    acc_sh.at[pl.ds(625 * s + 125 * (k + 1), 125)], nxt,
                    ssem).wait()
            out_cp.wait()

    return agg


_agg64 = _make_agg(64, feat_split=True)
_agg16 = _make_agg(16, feat_split=False)


def _mm_scale_body(x_ref, w_ref, deg_ref, o_ref):
    dinv = lax.rsqrt(deg_ref[...] + 1.0)
    val = jnp.dot(x_ref[...], w_ref[...],
                  preferred_element_type=jnp.float32) * dinv
    o_ref[...] = jnp.stack([val[:, :64], val[:, 64:]], axis=0)


def _layer2_body(p_ref, u_ref, deg_ref, b1_ref, w2_ref, o_ref):
    dinv = lax.rsqrt(deg_ref[...] + 1.0)
    pp = p_ref[...]
    uu = u_ref[...]
    agg = jnp.concatenate([pp[0] + uu[0], pp[1] + uu[1]], axis=1)
    h = jnp.maximum(agg * dinv + b1_ref[...], 0.0)
    o_ref[...] = jnp.dot(h, w2_ref[...],
                         preferred_element_type=jnp.float32) * dinv


def _final_body(r_ref, u2_ref, deg_ref, b2_ref, o_ref):
    dinv = lax.rsqrt(deg_ref[...] + 1.0)
    rr = r_ref[...]
    val = (rr[0] + rr[1] + u2_ref[...]) * dinv + b2_ref[...]
    o_ref[...] = val[:, :3]


def kernel(x, edge_index, W1, b1, W2, b2):
    f32 = jnp.float32
    ei = edge_index.astype(jnp.int32)
    src2 = ei[0].reshape(ROWS, CHUNK)
    dst2 = ei[1].reshape(ROWS, CHUNK)

    ones125 = jnp.ones((CHUNK,), f32)
    z640 = jnp.zeros((640,), f32)
    z64 = jnp.zeros((CHUNK, 64), f32)
    z16 = jnp.zeros((CHUNK, 16), f32)

    deg = _k1_deg(dst2, ones125, z640)
    deg2d = deg[:N_NODES, None]

    u = pl.pallas_call(
        _mm_scale_body,
        grid=(10,),
        in_specs=[
            pl.BlockSpec((1000, 500), lambda i: (i, 0)),
            pl.BlockSpec((500, 128), lambda i: (0, 0)),
            pl.BlockSpec((1000, 1), lambda i: (i, 0)),
        ],
        out_specs=pl.BlockSpec((2, 1000, 64), lambda i: (0, i, 0)),
        out_shape=jax.ShapeDtypeStruct((NC, N_NODES, 64), f32),
    )(x, W1, deg2d)

    p = _agg64(u, src2, dst2, z64)

    W2pad = jnp.zeros((128, 16), f32).at[:, :3].set(W2)
    u2 = pl.pallas_call(
        _layer2_body,
        grid=(10,),
        in_specs=[
            pl.BlockSpec((2, 1000, 64), lambda i: (0, i, 0)),
            pl.BlockSpec((2, 1000, 64), lambda i: (0, i, 0)),
            pl.BlockSpec((1000, 1), lambda i: (i, 0)),
            pl.BlockSpec((1, 128), lambda i: (0, 0)),
            pl.BlockSpec((128, 16), lambda i: (0, 0)),
        ],
        out_specs=pl.BlockSpec((1000, 16), lambda i: (i, 0)),
        out_shape=jax.ShapeDtypeStruct((N_NODES, 16), f32),
    )(p, u, deg2d, b1[None, :], W2pad)

    r = _agg16(u2, src2, dst2, z16)

    b2pad = jnp.zeros((16,), f32).at[:3].set(b2)
    outp = pl.pallas_call(
        _final_body,
        grid=(5,),
        in_specs=[
            pl.BlockSpec((2, 2000, 16), lambda i: (0, i, 0)),
            pl.BlockSpec((2000, 16), lambda i: (i, 0)),
            pl.BlockSpec((2000, 1), lambda i: (i, 0)),
            pl.BlockSpec((1, 16), lambda i: (0, 0)),
        ],
        out_specs=pl.BlockSpec((2000, 3), lambda i: (i, 0)),
        out_shape=jax.ShapeDtypeStruct((N_NODES, 3), f32),
    )(r, u2, deg2d, b2pad[None, :])

    return outp


# speedup vs baseline: 21.6398x; 1.0093x over previous
import functools

import jax
import jax.numpy as jnp
from jax import lax
from jax.experimental import pallas as pl
from jax.experimental.pallas import tpu as pltpu
from jax.experimental.pallas import tpu_sc as plsc

N_NODES = 10000
N_EDGES = 160000
N_PAD = 10240
CHUNK = 125
ROWS = N_EDGES // CHUNK
NC = 2
NS = 16
NB = 5
AHEAD = 2
LAG = NB - AHEAD

_MESH = plsc.VectorSubcoreMesh(core_axis_name="c", subcore_axis_name="s")
_SC_PARAMS = pltpu.CompilerParams(use_tc_tiling_on_sc=False)


@functools.partial(
    pl.kernel,
    mesh=_MESH,
    compiler_params=_SC_PARAMS,
    out_type=jax.ShapeDtypeStruct((N_PAD,), jnp.float32),
    scratch_types=[
        pltpu.VMEM_SHARED((N_PAD,), jnp.float32),
        pltpu.VMEM((80, CHUNK), jnp.int32),
        pltpu.VMEM((CHUNK,), jnp.float32),
        pltpu.VMEM((640,), jnp.float32),
    ],
)
def _k1_deg(dst_hbm, ones_hbm, z640_hbm, deg_out, deg_sh, dstv, onesv, zv):
    c = lax.axis_index("c")
    s = lax.axis_index("s")
    pltpu.sync_copy(z640_hbm, zv)
    pltpu.sync_copy(zv, deg_sh.at[pl.ds(640 * s, 640)])
    pltpu.sync_copy(ones_hbm, onesv)
    pltpu.sync_copy(dst_hbm.at[pl.ds(80 * s, 80)], dstv)
    plsc.subcore_barrier()

    def step(j, carry):
        pltpu.sync_copy(onesv, deg_sh.at[dstv.at[j]], add=True)
        return carry

    lax.fori_loop(0, 80, step, 0)
    plsc.subcore_barrier()

    base = 5120 * c + 320 * s
    pltpu.sync_copy(deg_sh.at[pl.ds(base, 320)], zv.at[pl.ds(0, 320)])
    pltpu.sync_copy(zv.at[pl.ds(0, 320)], deg_out.at[pl.ds(base, 320)])


def _make_agg(width, feat_split):
    R = 80 if feat_split else 40
    G = R // NB

    scratch = [
        pltpu.VMEM_SHARED((N_NODES, width), jnp.float32),
        pltpu.VMEM((R, CHUNK), jnp.int32),
        pltpu.VMEM((R, CHUNK), jnp.int32),
    ]
    scratch += [pltpu.VMEM((CHUNK, width), jnp.float32) for _ in range(NB)]
    scratch += [pltpu.SemaphoreType.DMA, pltpu.SemaphoreType.DMA]

    @functools.partial(
        pl.kernel,
        mesh=_MESH,
        compiler_params=_SC_PARAMS,
        out_type=jax.ShapeDtypeStruct((NC, N_NODES, width), jnp.float32),
        scratch_types=scratch,
    )
    def agg(u_hbm, src_hbm, dst_hbm, zrows_hbm, p_out, acc_sh, srcv, dstv,
            r0, r1, r2, r3, r4, gsem, ssem):
        rows = [r0, r1, r2, r3, r4]
        c = lax.axis_index("c")
        s = lax.axis_index("s")
        u_ref = u_hbm.at[c] if feat_split else u_hbm
        pltpu.sync_copy(zrows_hbm, r0)
        for k in range(5):
            pltpu.sync_copy(r0, acc_sh.at[pl.ds(625 * s + 125 * k, 125)])
        rowbase = R * s if feat_split else 640 * c + R * s
        pltpu.sync_copy(src_hbm.at[pl.ds(rowbase, R)], srcv)
        pltpu.sync_copy(dst_hbm.at[pl.ds(rowbase, R)], dstv)
        plsc.subcore_barrier()

        def g_start(j, b):
            pltpu.async_copy(u_ref.at[srcv.at[j]], rows[b], gsem)

        def g_wait(j, b):
            pltpu.make_async_copy(u_ref.at[srcv.at[j]], rows[b], gsem).wait()

        def s_start(j, b):
            pltpu.async_copy(rows[b], acc_sh.at[dstv.at[j]], ssem, add=True)

        def s_wait(j, b):
            pltpu.make_async_copy(rows[b], acc_sh.at[dstv.at[j]], ssem).wait()

        for b in range(AHEAD):
            g_start(b, b)

        def step(g, carry):
            for b in range(NB):
                j = NB * g + b
                g_wait(j, b)
                s_start(j, b)
                nb = (b + AHEAD) % NB
                if b < LAG:
                    @pl.when(g >= 1)
                    def _():
                        s_wait(j - LAG, nb)

                    g_start(j + AHEAD, nb)
                else:
                    s_wait(j - LAG, nb)

                    @pl.when(g <= G - 2)
                    def _():
                        g_start(j + AHEAD, nb)
            return carry

        lax.fori_loop(0, G, step, 0)
        for i in range(LAG):
            s_wait(R - LAG + i, (R - LAG + i) % NB)
        plsc.subcore_barrier()
        pltpu.sync_copy(acc_sh.at[pl.ds(625 * s, 125)], r0)
        for k in range(5):
            buf = rows[k % 2]
            nxt = rows[(k + 1) % 2]
            out_cp = pltpu.async_copy(
                buf, p_out.at[c, pl.ds(625 * s + 125 * k, 125)], gsem)
            if k < 4:
                pltpu.async_copy(
                    acc_sh.at[pl.ds(625 * s + 125 * (k + 1), 125)], nxt,
                    ssem).wait()
            out_cp.wait()

    return agg


_agg64 = _make_agg(64, feat_split=True)
_agg16 = _make_agg(16, feat_split=False)


def _mm_scale_body(x_ref, w_ref, deg_ref, o_ref):
    dinv = lax.rsqrt(deg_ref[...] + 1.0)
    val = jnp.dot(x_ref[...], w_ref[...],
                  preferred_element_type=jnp.float32) * dinv
    o_ref[...] = jnp.stack([val[:, :64], val[:, 64:]], axis=0)


def _layer2_body(p_ref, u_ref, deg_ref, b1_ref, w2_ref, o_ref):
    dinv = lax.rsqrt(deg_ref[...] + 1.0)
    pp = p_ref[...]
    uu = u_ref[...]
    agg = jnp.concatenate([pp[0] + uu[0], pp[1] + uu[1]], axis=1)
    h = jnp.maximum(agg * dinv + b1_ref[...], 0.0)
    o_ref[...] = jnp.dot(h, w2_ref[...],
                         preferred_element_type=jnp.float32) * dinv


def _final_body(r_ref, u2_ref, deg_ref, b2_ref, o_ref):
    dinv = lax.rsqrt(deg_ref[...] + 1.0)
    rr = r_ref[...]
    val = (rr[0] + rr[1] + u2_ref[...]) * dinv + b2_ref[...]
    o_ref[...] = val[:, :3]


def kernel(x, edge_index, W1, b1, W2, b2):
    f32 = jnp.float32
    ei = edge_index.astype(jnp.int32)
    src2 = ei[0].reshape(ROWS, CHUNK)
    dst2 = ei[1].reshape(ROWS, CHUNK)

    ones125 = jnp.ones((CHUNK,), f32)
    z640 = jnp.zeros((640,), f32)
    z64 = jnp.zeros((CHUNK, 64), f32)
    z16 = jnp.zeros((CHUNK, 16), f32)

    deg = _k1_deg(dst2, ones125, z640)
    deg2d = deg[:, None]

    u = pl.pallas_call(
        _mm_scale_body,
        grid=(10,),
        in_specs=[
            pl.BlockSpec((1000, 500), lambda i: (i, 0)),
            pl.BlockSpec((500, 128), lambda i: (0, 0)),
            pl.BlockSpec((1000, 1), lambda i: (i, 0)),
        ],
        out_specs=pl.BlockSpec((2, 1000, 64), lambda i: (0, i, 0)),
        out_shape=jax.ShapeDtypeStruct((NC, N_NODES, 64), f32),
    )(x, W1, deg2d)

    p = _agg64(u, src2, dst2, z64)

    W2pad = jnp.zeros((128, 16), f32).at[:, :3].set(W2)
    u2 = pl.pallas_call(
        _layer2_body,
        grid=(10,),
        in_specs=[
            pl.BlockSpec((2, 1000, 64), lambda i: (0, i, 0)),
            pl.BlockSpec((2, 1000, 64), lambda i: (0, i, 0)),
            pl.BlockSpec((1000, 1), lambda i: (i, 0)),
            pl.BlockSpec((1, 128), lambda i: (0, 0)),
            pl.BlockSpec((128, 16), lambda i: (0, 0)),
        ],
        out_specs=pl.BlockSpec((1000, 16), lambda i: (i, 0)),
        out_shape=jax.ShapeDtypeStruct((N_NODES, 16), f32),
    )(p, u, deg2d, b1[None, :], W2pad)

    r = _agg16(u2, src2, dst2, z16)

    b2pad = jnp.zeros((16,), f32).at[:3].set(b2)
    outp = pl.pallas_call(
        _final_body,
        grid=(5,),
        in_specs=[
            pl.BlockSpec((2, 2000, 16), lambda i: (0, i, 0)),
            pl.BlockSpec((2000, 16), lambda i: (i, 0)),
            pl.BlockSpec((2000, 1), lambda i: (i, 0)),
            pl.BlockSpec((1, 16), lambda i: (0, 0)),
        ],
        out_specs=pl.BlockSpec((2000, 3), lambda i: (i, 0)),
        out_shape=jax.ShapeDtypeStruct((N_NODES, 3), f32),
    )(r, u2, deg2d, b2pad[None, :])

    return outp


# speedup vs baseline: 23.5276x; 1.0872x over previous
import functools

import jax
import jax.numpy as jnp
from jax import lax
from jax.experimental import pallas as pl
from jax.experimental.pallas import tpu as pltpu
from jax.experimental.pallas import tpu_sc as plsc

N_NODES = 10000
N_EDGES = 160000
N_PAD = 10240
CHUNK = 125
ROWS = N_EDGES // CHUNK
NC = 2
NS = 16
NB = 5
AHEAD = 2
LAG = NB - AHEAD

_MESH = plsc.VectorSubcoreMesh(core_axis_name="c", subcore_axis_name="s")
_SC_PARAMS = pltpu.CompilerParams(use_tc_tiling_on_sc=False)


@functools.partial(
    pl.kernel,
    mesh=_MESH,
    compiler_params=_SC_PARAMS,
    out_type=jax.ShapeDtypeStruct((N_PAD,), jnp.float32),
    scratch_types=[
        pltpu.VMEM_SHARED((N_PAD,), jnp.float32),
        pltpu.VMEM((80, CHUNK), jnp.int32),
        pltpu.VMEM((CHUNK,), jnp.float32),
        pltpu.VMEM((640,), jnp.float32),
    ],
)
def _k1_deg(dst_hbm, ones_hbm, z640_hbm, deg_out, deg_sh, dstv, onesv, zv):
    c = lax.axis_index("c")
    s = lax.axis_index("s")
    pltpu.sync_copy(z640_hbm, zv)
    pltpu.sync_copy(zv, deg_sh.at[pl.ds(640 * s, 640)])
    pltpu.sync_copy(ones_hbm, onesv)
    pltpu.sync_copy(dst_hbm.at[pl.ds(80 * s, 80)], dstv)
    plsc.subcore_barrier()

    def step(j, carry):
        pltpu.sync_copy(onesv, deg_sh.at[dstv.at[j]], add=True)
        return carry

    lax.fori_loop(0, 80, step, 0)
    plsc.subcore_barrier()

    base = 5120 * c + 320 * s
    pltpu.sync_copy(deg_sh.at[pl.ds(base, 320)], zv.at[pl.ds(0, 320)])
    pltpu.sync_copy(zv.at[pl.ds(0, 320)], deg_out.at[pl.ds(base, 320)])


def _make_agg(width, feat_split):
    R = 80 if feat_split else 40
    G = R // NB

    scratch = [
        pltpu.VMEM_SHARED((N_NODES, width), jnp.float32),
        pltpu.VMEM((R, CHUNK), jnp.int32),
        pltpu.VMEM((R, CHUNK), jnp.int32),
    ]
    scratch += [pltpu.VMEM((CHUNK, width), jnp.float32) for _ in range(NB)]
    scratch += [pltpu.SemaphoreType.DMA, pltpu.SemaphoreType.DMA]

    @functools.partial(
        pl.kernel,
        mesh=_MESH,
        compiler_params=_SC_PARAMS,
        out_type=jax.ShapeDtypeStruct((N_NODES, 128), jnp.float32),
        scratch_types=scratch,
    )
    def agg(u_hbm, src_hbm, dst_hbm, zrows_hbm, p_out, acc_sh, srcv, dstv,
            r0, r1, r2, r3, r4, gsem, ssem):
        rows = [r0, r1, r2, r3, r4]
        c = lax.axis_index("c")
        s = lax.axis_index("s")
        pltpu.sync_copy(zrows_hbm, r0)
        for k in range(5):
            pltpu.sync_copy(r0, acc_sh.at[pl.ds(625 * s + 125 * k, 125)])
        rowbase = R * s if feat_split else 640 * c + R * s
        pltpu.sync_copy(src_hbm.at[pl.ds(rowbase, R)], srcv)
        pltpu.sync_copy(dst_hbm.at[pl.ds(rowbase, R)], dstv)
        plsc.subcore_barrier()

        u_ref = u_hbm.at[c] if feat_split else u_hbm

        def g_start(j, b):
            pltpu.async_copy(u_ref.at[srcv.at[j]], rows[b], gsem)

        def g_wait(j, b):
            pltpu.make_async_copy(u_ref.at[srcv.at[j]], rows[b], gsem).wait()

        def s_start(j, b):
            pltpu.async_copy(rows[b], acc_sh.at[dstv.at[j]], ssem, add=True)

        def s_wait(j, b):
            pltpu.make_async_copy(rows[b], acc_sh.at[dstv.at[j]], ssem).wait()

        for b in range(AHEAD):
            g_start(b, b)

        def step(g, carry):
            for b in range(NB):
                j = NB * g + b
                g_wait(j, b)
                s_start(j, b)
                nb = (b + AHEAD) % NB
                if b < LAG:
                    @pl.when(g >= 1)
                    def _():
                        s_wait(j - LAG, nb)

                    g_start(j + AHEAD, nb)
                else:
                    s_wait(j - LAG, nb)

                    @pl.when(g <= G - 2)
                    def _():
                        g_start(j + AHEAD, nb)
            return carry

        lax.fori_loop(0, G, step, 0)
        for i in range(LAG):
            s_wait(R - LAG + i, (R - LAG + i) % NB)
        plsc.subcore_barrier()
        pcol = width * c
        pltpu.sync_copy(acc_sh.at[pl.ds(625 * s, 125)], r0)
        for k in range(5):
            buf = rows[k % 2]
            nxt = rows[(k + 1) % 2]
            out_cp = pltpu.async_copy(
                buf,
                p_out.at[pl.ds(625 * s + 125 * k, 125), pl.ds(pcol, width)],
                gsem)
            if k < 4:
                pltpu.async_copy(
                    acc_sh.at[pl.ds(625 * s + 125 * (k + 1), 125)], nxt,
                    ssem).wait()
            out_cp.wait()

    return agg


_agg64 = _make_agg(64, feat_split=True)
_agg16 = _make_agg(16, feat_split=False)


def _mm_scale_body(x_ref, w_ref, deg_ref, o_ref):
    dinv = lax.rsqrt(deg_ref[...] + 1.0)
    val = jnp.dot(x_ref[...], w_ref[...],
                  preferred_element_type=jnp.float32) * dinv
    o_ref[...] = jnp.stack([val[:, :64], val[:, 64:]], axis=0)


def _layer2_body(p_ref, u_ref, deg_ref, b1_ref, w2_ref, o_ref):
    dinv = lax.rsqrt(deg_ref[...] + 1.0)
    uu = u_ref[...]
    agg = p_ref[...] + jnp.concatenate([uu[0], uu[1]], axis=1)
    h = jnp.maximum(agg * dinv + b1_ref[...], 0.0)
    o_ref[...] = jnp.dot(h, w2_ref[...],
                         preferred_element_type=jnp.float32) * dinv


def _final_body(r_ref, u2_ref, deg_ref, b2_ref, o_ref):
    dinv = lax.rsqrt(deg_ref[...] + 1.0)
    rr = r_ref[...]
    val = (rr[:, :16] + rr[:, 16:32] + u2_ref[...]) * dinv + b2_ref[...]
    o_ref[...] = val[:, :3]


def kernel(x, edge_index, W1, b1, W2, b2):
    f32 = jnp.float32
    ei = edge_index.astype(jnp.int32)
    src2 = ei[0].reshape(ROWS, CHUNK)
    dst2 = ei[1].reshape(ROWS, CHUNK)

    ones125 = jnp.ones((CHUNK,), f32)
    z640 = jnp.zeros((640,), f32)
    z64 = jnp.zeros((CHUNK, 64), f32)
    z16 = jnp.zeros((CHUNK, 16), f32)

    deg = _k1_deg(dst2, ones125, z640)
    deg2d = deg[:, None]

    u = pl.pallas_call(
        _mm_scale_body,
        grid=(10,),
        in_specs=[
            pl.BlockSpec((1000, 500), lambda i: (i, 0)),
            pl.BlockSpec((500, 128), lambda i: (0, 0)),
            pl.BlockSpec((1000, 1), lambda i: (i, 0)),
        ],
        out_specs=pl.BlockSpec((2, 1000, 64), lambda i: (0, i, 0)),
        out_shape=jax.ShapeDtypeStruct((NC, N_NODES, 64), f32),
    )(x, W1, deg2d)

    p = _agg64(u, src2, dst2, z64)

    W2pad = jnp.zeros((128, 16), f32).at[:, :3].set(W2)
    u2 = pl.pallas_call(
        _layer2_body,
        grid=(10,),
        in_specs=[
            pl.BlockSpec((1000, 128), lambda i: (i, 0)),
            pl.BlockSpec((2, 1000, 64), lambda i: (0, i, 0)),
            pl.BlockSpec((1000, 1), lambda i: (i, 0)),
            pl.BlockSpec((1, 128), lambda i: (0, 0)),
            pl.BlockSpec((128, 16), lambda i: (0, 0)),
        ],
        out_specs=pl.BlockSpec((1000, 16), lambda i: (i, 0)),
        out_shape=jax.ShapeDtypeStruct((N_NODES, 16), f32),
    )(p, u, deg2d, b1[None, :], W2pad)

    r = _agg16(u2, src2, dst2, z16)

    b2pad = jnp.zeros((16,), f32).at[:3].set(b2)
    outp = pl.pallas_call(
        _final_body,
        grid=(5,),
        in_specs=[
            pl.BlockSpec((2000, 128), lambda i: (i, 0)),
            pl.BlockSpec((2000, 16), lambda i: (i, 0)),
            pl.BlockSpec((2000, 1), lambda i: (i, 0)),
            pl.BlockSpec((1, 16), lambda i: (0, 0)),
        ],
        out_specs=pl.BlockSpec((2000, 3), lambda i: (i, 0)),
        out_shape=jax.ShapeDtypeStruct((N_NODES, 3), f32),
    )(r, u2, deg2d, b2pad[None, :])

    return outp


# speedup vs baseline: 25.9201x; 1.1017x over previous
import functools

import jax
import jax.numpy as jnp
from jax import lax
from jax.experimental import pallas as pl
from jax.experimental.pallas import tpu as pltpu
from jax.experimental.pallas import tpu_sc as plsc

N_NODES = 10000
N_EDGES = 160000
N_PAD = 10240
CHUNK = 125
ROWS = N_EDGES // CHUNK
NC = 2
NS = 16
NB = 8
AHEAD = 4
LAG = NB - AHEAD

_MESH = plsc.VectorSubcoreMesh(core_axis_name="c", subcore_axis_name="s")
_SC_PARAMS = pltpu.CompilerParams(use_tc_tiling_on_sc=False)


@functools.partial(
    pl.kernel,
    mesh=_MESH,
    compiler_params=_SC_PARAMS,
    out_type=jax.ShapeDtypeStruct((N_PAD,), jnp.float32),
    scratch_types=[
        pltpu.VMEM_SHARED((N_PAD,), jnp.float32),
        pltpu.VMEM((80, CHUNK), jnp.int32),
        pltpu.VMEM((CHUNK,), jnp.float32),
        pltpu.VMEM((640,), jnp.float32),
    ],
)
def _k1_deg(dst_hbm, ones_hbm, z640_hbm, deg_out, deg_sh, dstv, onesv, zv):
    c = lax.axis_index("c")
    s = lax.axis_index("s")
    pltpu.sync_copy(z640_hbm, zv)
    pltpu.sync_copy(zv, deg_sh.at[pl.ds(640 * s, 640)])
    pltpu.sync_copy(ones_hbm, onesv)
    pltpu.sync_copy(dst_hbm.at[pl.ds(80 * s, 80)], dstv)
    plsc.subcore_barrier()

    def step(j, carry):
        pltpu.sync_copy(onesv, deg_sh.at[dstv.at[j]], add=True)
        return carry

    lax.fori_loop(0, 80, step, 0)
    plsc.subcore_barrier()

    base = 5120 * c + 320 * s
    pltpu.sync_copy(deg_sh.at[pl.ds(base, 320)], zv.at[pl.ds(0, 320)])
    pltpu.sync_copy(zv.at[pl.ds(0, 320)], deg_out.at[pl.ds(base, 320)])


def _make_agg(width, feat_split):
    R = 80 if feat_split else 40
    G = R // NB

    scratch = [
        pltpu.VMEM_SHARED((N_NODES, width), jnp.float32),
        pltpu.VMEM((R, CHUNK), jnp.int32),
        pltpu.VMEM((R, CHUNK), jnp.int32),
    ]
    scratch += [pltpu.VMEM((CHUNK, width), jnp.float32) for _ in range(NB)]
    scratch += [pltpu.SemaphoreType.DMA, pltpu.SemaphoreType.DMA]

    @functools.partial(
        pl.kernel,
        mesh=_MESH,
        compiler_params=_SC_PARAMS,
        out_type=jax.ShapeDtypeStruct((N_NODES, 128), jnp.float32),
        scratch_types=scratch,
    )
    def agg(u_hbm, src_hbm, dst_hbm, zrows_hbm, p_out, acc_sh, srcv, dstv,
            r0, r1, r2, r3, r4, r5, r6, r7, gsem, ssem):
        rows = [r0, r1, r2, r3, r4, r5, r6, r7]
        c = lax.axis_index("c")
        s = lax.axis_index("s")
        pltpu.sync_copy(zrows_hbm, r0)
        for k in range(5):
            pltpu.sync_copy(r0, acc_sh.at[pl.ds(625 * s + 125 * k, 125)])
        rowbase = R * s if feat_split else 640 * c + R * s
        pltpu.sync_copy(src_hbm.at[pl.ds(rowbase, R)], srcv)
        pltpu.sync_copy(dst_hbm.at[pl.ds(rowbase, R)], dstv)
        plsc.subcore_barrier()

        u_ref = u_hbm.at[c] if feat_split else u_hbm

        def g_start(j, b):
            pltpu.async_copy(u_ref.at[srcv.at[j]], rows[b], gsem)

        def g_wait(j, b):
            pltpu.make_async_copy(u_ref.at[srcv.at[j]], rows[b], gsem).wait()

        def s_start(j, b):
            pltpu.async_copy(rows[b], acc_sh.at[dstv.at[j]], ssem, add=True)

        def s_wait(j, b):
            pltpu.make_async_copy(rows[b], acc_sh.at[dstv.at[j]], ssem).wait()

        for b in range(AHEAD):
            g_start(b, b)

        def step(g, carry):
            for b in range(NB):
                j = NB * g + b
                g_wait(j, b)
                s_start(j, b)
                nb = (b + AHEAD) % NB
                if b < LAG:
                    @pl.when(g >= 1)
                    def _():
                        s_wait(j - LAG, nb)

                    g_start(j + AHEAD, nb)
                else:
                    s_wait(j - LAG, nb)

                    @pl.when(g <= G - 2)
                    def _():
                        g_start(j + AHEAD, nb)
            return carry

        lax.fori_loop(0, G, step, 0)
        for i in range(LAG):
            s_wait(R - LAG + i, (R - LAG + i) % NB)
        plsc.subcore_barrier()
        pcol = width * c
        pltpu.sync_copy(acc_sh.at[pl.ds(625 * s, 125)], r0)
        for k in range(5):
            buf = rows[k % 2]
            nxt = rows[(k + 1) % 2]
            out_cp = pltpu.async_copy(
                buf,
                p_out.at[pl.ds(625 * s + 125 * k, 125), pl.ds(pcol, width)],
                gsem)
            if k < 4:
                pltpu.async_copy(
                    acc_sh.at[pl.ds(625 * s + 125 * (k + 1), 125)], nxt,
                    ssem).wait()
            out_cp.wait()

    return agg


_agg64 = _make_agg(64, feat_split=True)
_agg16 = _make_agg(16, feat_split=False)


def _mm_scale_body(x_ref, w_ref, deg_ref, o_ref):
    dinv = lax.rsqrt(deg_ref[...] + 1.0)
    val = jnp.dot(x_ref[...], w_ref[...],
                  preferred_element_type=jnp.float32) * dinv
    o_ref[...] = jnp.stack([val[:, :64], val[:, 64:]], axis=0)


def _layer2_body(p_ref, u_ref, deg_ref, b1_ref, w2_ref, o_ref):
    dinv = lax.rsqrt(deg_ref[...] + 1.0)
    uu = u_ref[...]
    agg = p_ref[...] + jnp.concatenate([uu[0], uu[1]], axis=1)
    h = jnp.maximum(agg * dinv + b1_ref[...], 0.0)
    o_ref[...] = jnp.dot(h, w2_ref[...],
                         preferred_element_type=jnp.float32) * dinv


def _final_body(r_ref, u2_ref, deg_ref, b2_ref, o_ref):
    dinv = lax.rsqrt(deg_ref[...] + 1.0)
    rr = r_ref[...]
    val = (rr[:, :16] + rr[:, 16:32] + u2_ref[...]) * dinv + b2_ref[...]
    o_ref[...] = val[:, :3]


def kernel(x, edge_index, W1, b1, W2, b2):
    f32 = jnp.float32
    ei = edge_index.astype(jnp.int32)
    src2 = ei[0].reshape(ROWS, CHUNK)
    dst2 = ei[1].reshape(ROWS, CHUNK)

    ones125 = jnp.ones((CHUNK,), f32)
    z640 = jnp.zeros((640,), f32)
    z64 = jnp.zeros((CHUNK, 64), f32)
    z16 = jnp.zeros((CHUNK, 16), f32)

    deg = _k1_deg(dst2, ones125, z640)
    deg2d = deg[:, None]

    u = pl.pallas_call(
        _mm_scale_body,
        grid=(10,),
        in_specs=[
            pl.BlockSpec((1000, 500), lambda i: (i, 0)),
            pl.BlockSpec((500, 128), lambda i: (0, 0)),
            pl.BlockSpec((1000, 1), lambda i: (i, 0)),
        ],
        out_specs=pl.BlockSpec((2, 1000, 64), lambda i: (0, i, 0)),
        out_shape=jax.ShapeDtypeStruct((NC, N_NODES, 64), f32),
    )(x, W1, deg2d)

    p = _agg64(u, src2, dst2, z64)

    W2pad = jnp.zeros((128, 16), f32).at[:, :3].set(W2)
    u2 = pl.pallas_call(
        _layer2_body,
        grid=(10,),
        in_specs=[
            pl.BlockSpec((1000, 128), lambda i: (i, 0)),
            pl.BlockSpec((2, 1000, 64), lambda i: (0, i, 0)),
            pl.BlockSpec((1000, 1), lambda i: (i, 0)),
            pl.BlockSpec((1, 128), lambda i: (0, 0)),
            pl.BlockSpec((128, 16), lambda i: (0, 0)),
        ],
        out_specs=pl.BlockSpec((1000, 16), lambda i: (i, 0)),
        out_shape=jax.ShapeDtypeStruct((N_NODES, 16), f32),
    )(p, u, deg2d, b1[None, :], W2pad)

    r = _agg16(u2, src2, dst2, z16)

    b2pad = jnp.zeros((16,), f32).at[:3].set(b2)
    outp = pl.pallas_call(
        _final_body,
        grid=(5,),
        in_specs=[
            pl.BlockSpec((2000, 128), lambda i: (i, 0)),
            pl.BlockSpec((2000, 16), lambda i: (i, 0)),
            pl.BlockSpec((2000, 1), lambda i: (i, 0)),
            pl.BlockSpec((1, 16), lambda i: (0, 0)),
        ],
        out_specs=pl.BlockSpec((2000, 3), lambda i: (i, 0)),
        out_shape=jax.ShapeDtypeStruct((N_NODES, 3), f32),
    )(r, u2, deg2d, b2pad[None, :])

    return outp


# speedup vs baseline: 26.3407x; 1.0162x over previous
import functools

import jax
import jax.numpy as jnp
from jax import lax
from jax.experimental import pallas as pl
from jax.experimental.pallas import tpu as pltpu
from jax.experimental.pallas import tpu_sc as plsc

N_NODES = 10000
N_EDGES = 160000
N_PAD = 10240
CHUNK = 125
ROWS = N_EDGES // CHUNK
NC = 2
NS = 16

_MESH = plsc.VectorSubcoreMesh(core_axis_name="c", subcore_axis_name="s")
_SC_PARAMS = pltpu.CompilerParams(use_tc_tiling_on_sc=False)


@functools.partial(
    pl.kernel,
    mesh=_MESH,
    compiler_params=_SC_PARAMS,
    out_type=jax.ShapeDtypeStruct((N_PAD,), jnp.float32),
    scratch_types=[
        pltpu.VMEM_SHARED((N_PAD,), jnp.float32),
        pltpu.VMEM((80, CHUNK), jnp.int32),
        pltpu.VMEM((CHUNK,), jnp.float32),
        pltpu.VMEM((640,), jnp.float32),
    ],
)
def _k1_deg(dst_hbm, ones_hbm, z640_hbm, deg_out, deg_sh, dstv, onesv, zv):
    c = lax.axis_index("c")
    s = lax.axis_index("s")
    pltpu.sync_copy(z640_hbm, zv)
    pltpu.sync_copy(zv, deg_sh.at[pl.ds(640 * s, 640)])
    pltpu.sync_copy(ones_hbm, onesv)
    pltpu.sync_copy(dst_hbm.at[pl.ds(80 * s, 80)], dstv)
    plsc.subcore_barrier()

    def step(j, carry):
        pltpu.sync_copy(onesv, deg_sh.at[dstv.at[j]], add=True)
        return carry

    lax.fori_loop(0, 80, step, 0)
    plsc.subcore_barrier()

    base = 5120 * c + 320 * s
    pltpu.sync_copy(deg_sh.at[pl.ds(base, 320)], zv.at[pl.ds(0, 320)])
    pltpu.sync_copy(zv.at[pl.ds(0, 320)], deg_out.at[pl.ds(base, 320)])


def _make_agg(width, feat_split, NB):
    AHEAD = NB // 2
    LAG = NB - AHEAD
    R = 80 if feat_split else 40
    G = R // NB

    scratch = [
        pltpu.VMEM_SHARED((N_NODES, width), jnp.float32),
        pltpu.VMEM((R, CHUNK), jnp.int32),
        pltpu.VMEM((R, CHUNK), jnp.int32),
    ]
    scratch += [pltpu.VMEM((CHUNK, width), jnp.float32) for _ in range(NB)]
    scratch += [pltpu.SemaphoreType.DMA, pltpu.SemaphoreType.DMA]

    @functools.partial(
        pl.kernel,
        mesh=_MESH,
        compiler_params=_SC_PARAMS,
        out_type=jax.ShapeDtypeStruct((N_NODES, 128), jnp.float32),
        scratch_types=scratch,
    )
    def agg(u_hbm, src_hbm, dst_hbm, zrows_hbm, p_out, acc_sh, srcv, dstv,
            *bufs_and_sems):
        rows = list(bufs_and_sems[:NB])
        gsem, ssem = bufs_and_sems[NB], bufs_and_sems[NB + 1]
        r0 = rows[0]
        c = lax.axis_index("c")
        s = lax.axis_index("s")
        pltpu.sync_copy(zrows_hbm, r0)
        for k in range(5):
            pltpu.sync_copy(r0, acc_sh.at[pl.ds(625 * s + 125 * k, 125)])
        rowbase = R * s if feat_split else 640 * c + R * s
        pltpu.sync_copy(src_hbm.at[pl.ds(rowbase, R)], srcv)
        pltpu.sync_copy(dst_hbm.at[pl.ds(rowbase, R)], dstv)
        plsc.subcore_barrier()

        u_ref = u_hbm.at[c] if feat_split else u_hbm

        def g_start(j, b):
            pltpu.async_copy(u_ref.at[srcv.at[j]], rows[b], gsem)

        def g_wait(j, b):
            pltpu.make_async_copy(u_ref.at[srcv.at[j]], rows[b], gsem).wait()

        def s_start(j, b):
            pltpu.async_copy(rows[b], acc_sh.at[dstv.at[j]], ssem, add=True)

        def s_wait(j, b):
            pltpu.make_async_copy(rows[b], acc_sh.at[dstv.at[j]], ssem).wait()

        for b in range(AHEAD):
            g_start(b, b)

        def step(g, carry):
            for b in range(NB):
                j = NB * g + b
                g_wait(j, b)
                s_start(j, b)
                nb = (b + AHEAD) % NB
                if b < LAG:
                    @pl.when(g >= 1)
                    def _():
                        s_wait(j - LAG, nb)

                    g_start(j + AHEAD, nb)
                else:
                    s_wait(j - LAG, nb)

                    @pl.when(g <= G - 2)
                    def _():
                        g_start(j + AHEAD, nb)
            return carry

        lax.fori_loop(0, G, step, 0)
        for i in range(LAG):
            s_wait(R - LAG + i, (R - LAG + i) % NB)
        plsc.subcore_barrier()
        pcol = width * c
        pltpu.sync_copy(acc_sh.at[pl.ds(625 * s, 125)], r0)
        for k in range(5):
            buf = rows[k % 2]
            nxt = rows[(k + 1) % 2]
            out_cp = pltpu.async_copy(
                buf,
                p_out.at[pl.ds(625 * s + 125 * k, 125), pl.ds(pcol, width)],
                gsem)
            if k < 4:
                pltpu.async_copy(
                    acc_sh.at[pl.ds(625 * s + 125 * (k + 1), 125)], nxt,
                    ssem).wait()
            out_cp.wait()

    return agg


_agg64 = _make_agg(64, feat_split=True, NB=8)
_agg16 = _make_agg(16, feat_split=False, NB=10)


def _mm_scale_body(x_ref, w_ref, deg_ref, o_ref):
    dinv = lax.rsqrt(deg_ref[...] + 1.0)
    val = jnp.dot(x_ref[...], w_ref[...],
                  preferred_element_type=jnp.float32) * dinv
    o_ref[...] = jnp.stack([val[:, :64], val[:, 64:]], axis=0)


def _layer2_body(p_ref, u_ref, deg_ref, b1_ref, w2_ref, o_ref):
    dinv = lax.rsqrt(deg_ref[...] + 1.0)
    uu = u_ref[...]
    agg = p_ref[...] + jnp.concatenate([uu[0], uu[1]], axis=1)
    h = jnp.maximum(agg * dinv + b1_ref[...], 0.0)
    o_ref[...] = jnp.dot(h, w2_ref[...],
                         preferred_element_type=jnp.float32) * dinv


def _final_body(r_ref, u2_ref, deg_ref, b2_ref, o_ref):
    dinv = lax.rsqrt(deg_ref[...] + 1.0)
    rr = r_ref[...]
    val = (rr[:, :16] + rr[:, 16:32] + u2_ref[...]) * dinv + b2_ref[...]
    o_ref[...] = val[:, :3]


def kernel(x, edge_index, W1, b1, W2, b2):
    f32 = jnp.float32
    ei = edge_index.astype(jnp.int32)
    src2 = ei[0].reshape(ROWS, CHUNK)
    dst2 = ei[1].reshape(ROWS, CHUNK)

    ones125 = jnp.ones((CHUNK,), f32)
    z640 = jnp.zeros((640,), f32)
    z64 = jnp.zeros((CHUNK, 64), f32)
    z16 = jnp.zeros((CHUNK, 16), f32)

    deg = _k1_deg(dst2, ones125, z640)
    deg2d = deg[:, None]

    u = pl.pallas_call(
        _mm_scale_body,
        grid=(10,),
        in_specs=[
            pl.BlockSpec((1000, 500), lambda i: (i, 0)),
            pl.BlockSpec((500, 128), lambda i: (0, 0)),
            pl.BlockSpec((1000, 1), lambda i: (i, 0)),
        ],
        out_specs=pl.BlockSpec((2, 1000, 64), lambda i: (0, i, 0)),
        out_shape=jax.ShapeDtypeStruct((NC, N_NODES, 64), f32),
    )(x, W1, deg2d)

    p = _agg64(u, src2, dst2, z64)

    W2pad = jnp.zeros((128, 16), f32).at[:, :3].set(W2)
    u2 = pl.pallas_call(
        _layer2_body,
        grid=(10,),
        in_specs=[
            pl.BlockSpec((1000, 128), lambda i: (i, 0)),
            pl.BlockSpec((2, 1000, 64), lambda i: (0, i, 0)),
            pl.BlockSpec((1000, 1), lambda i: (i, 0)),
            pl.BlockSpec((1, 128), lambda i: (0, 0)),
            pl.BlockSpec((128, 16), lambda i: (0, 0)),
        ],
        out_specs=pl.BlockSpec((1000, 16), lambda i: (i, 0)),
        out_shape=jax.ShapeDtypeStruct((N_NODES, 16), f32),
    )(p, u, deg2d, b1[None, :], W2pad)

    r = _agg16(u2, src2, dst2, z16)

    b2pad = jnp.zeros((16,), f32).at[:3].set(b2)
    outp = pl.pallas_call(
        _final_body,
        grid=(5,),
        in_specs=[
            pl.BlockSpec((2000, 128), lambda i: (i, 0)),
            pl.BlockSpec((2000, 16), lambda i: (i, 0)),
            pl.BlockSpec((2000, 1), lambda i: (i, 0)),
            pl.BlockSpec((1, 16), lambda i: (0, 0)),
        ],
        out_specs=pl.BlockSpec((2000, 3), lambda i: (i, 0)),
        out_shape=jax.ShapeDtypeStruct((N_NODES, 3), f32),
    )(r, u2, deg2d, b2pad[None, :])

    return outp
